# baseline (speedup 1.0000x reference)
import functools

import jax
import jax.numpy as jnp
from jax import lax
from jax.experimental import pallas as pl
from jax.experimental.pallas import tpu as pltpu
from jax.experimental.pallas import tpu_sc as plsc

_N_CAT = 26
_N_CONT = 13
_DIM = 32
_BATCH = 16384
_NC = 2
_NS = 16
_NW = _NC * _NS
_BPW = _BATCH // _NW


def _emb_kernel(idx_hbm, *rest):
    tabs = rest[:_N_CAT]
    emb_out, idx_v, rows_v, sem = rest[_N_CAT:]
    wid = lax.axis_index("s") * _NC + lax.axis_index("c")
    base = wid * _BPW

    for i in range(_N_CAT):
        pltpu.sync_copy(idx_hbm.at[pl.ds(i * _BATCH + base, _BPW)], idx_v)
        pltpu.async_copy(tabs[i].at[idx_v], rows_v, sem).wait()
        pltpu.sync_copy(rows_v, emb_out.at[pl.ds(i * _BATCH + base, _BPW), :])


@jax.jit
def _run(idx, *tabs):
    mesh = plsc.VectorSubcoreMesh(core_axis_name="c", subcore_axis_name="s")
    f = functools.partial(
        pl.kernel,
        out_type=jax.ShapeDtypeStruct((_N_CAT * _BATCH, _DIM), jnp.float32),
        mesh=mesh,
        scratch_types=[
            pltpu.VMEM((_BPW,), jnp.int32),
            pltpu.VMEM((_BPW, _DIM), jnp.float32),
            pltpu.SemaphoreType.DMA,
        ],
        compiler_params=pltpu.CompilerParams(use_tc_tiling_on_sc=False),
    )(_emb_kernel)
    emb = f(idx, *tabs)
    return emb.reshape(_N_CAT, _BATCH, _DIM).transpose(1, 0, 2).reshape(
        _BATCH, _N_CAT * _DIM)


def kernel(x, table_0, table_1, table_2, table_3, table_4, table_5, table_6,
           table_7, table_8, table_9, table_10, table_11, table_12, table_13,
           table_14, table_15, table_16, table_17, table_18, table_19,
           table_20, table_21, table_22, table_23, table_24, table_25):
    tabs = (table_0, table_1, table_2, table_3, table_4, table_5, table_6,
            table_7, table_8, table_9, table_10, table_11, table_12, table_13,
            table_14, table_15, table_16, table_17, table_18, table_19,
            table_20, table_21, table_22, table_23, table_24, table_25)
    idx = x[:, :_N_CAT].astype(jnp.int32).T.reshape(-1)
    emb = _run(idx, *tabs)
    cont = x[:, _N_CAT:_N_CAT + _N_CONT]
    return emb, cont


# speedup vs baseline: 1.0912x; 1.0912x over previous
import functools

import jax
import jax.numpy as jnp
from jax import lax
from jax.experimental import pallas as pl
from jax.experimental.pallas import tpu as pltpu
from jax.experimental.pallas import tpu_sc as plsc

_N_CAT = 26
_N_CONT = 13
_DIM = 32
_BATCH = 16384
_NC = 2
_NS = 16
_NW = _NC * _NS
_BPW = _BATCH // _NW


def _emb_kernel(idx_hbm, *rest):
    tabs = rest[:_N_CAT]
    emb_out = rest[_N_CAT]
    idx_v = rest[_N_CAT + 1]
    rows = rest[_N_CAT + 2:_N_CAT + 4]
    gsem = rest[_N_CAT + 4:_N_CAT + 6]
    wsem = rest[_N_CAT + 6:_N_CAT + 8]
    wid = lax.axis_index("s") * _NC + lax.axis_index("c")
    base = wid * _BPW

    pltpu.sync_copy(idx_hbm.at[:, pl.ds(base, _BPW)], idx_v)

    gd = [None, None]
    wd = [None, None]
    for i in range(_N_CAT):
        b = i % 2
        if wd[b] is not None:
            wd[b].wait()
        gd[b] = pltpu.async_copy(tabs[i].at[idx_v.at[i]], rows[b], gsem[b])
        gd[b].wait()
        wd[b] = pltpu.async_copy(
            rows[b], emb_out.at[pl.ds(base, _BPW), pl.ds(i * _DIM, _DIM)],
            wsem[b])
    wd[0].wait()
    wd[1].wait()


@jax.jit
def _run(idx, *tabs):
    mesh = plsc.VectorSubcoreMesh(core_axis_name="c", subcore_axis_name="s")
    f = functools.partial(
        pl.kernel,
        out_type=jax.ShapeDtypeStruct((_BATCH, _N_CAT * _DIM), jnp.float32),
        mesh=mesh,
        scratch_types=[
            pltpu.VMEM((_N_CAT, _BPW), jnp.int32),
            pltpu.VMEM((_BPW, _DIM), jnp.float32),
            pltpu.VMEM((_BPW, _DIM), jnp.float32),
            pltpu.SemaphoreType.DMA,
            pltpu.SemaphoreType.DMA,
            pltpu.SemaphoreType.DMA,
            pltpu.SemaphoreType.DMA,
        ],
        compiler_params=pltpu.CompilerParams(use_tc_tiling_on_sc=False),
    )(_emb_kernel)
    return f(idx, *tabs)


def kernel(x, table_0, table_1, table_2, table_3, table_4, table_5, table_6,
           table_7, table_8, table_9, table_10, table_11, table_12, table_13,
           table_14, table_15, table_16, table_17, table_18, table_19,
           table_20, table_21, table_22, table_23, table_24, table_25):
    tabs = (table_0, table_1, table_2, table_3, table_4, table_5, table_6,
            table_7, table_8, table_9, table_10, table_11, table_12, table_13,
            table_14, table_15, table_16, table_17, table_18, table_19,
            table_20, table_21, table_22, table_23, table_24, table_25)
    idx = x[:, :_N_CAT].astype(jnp.int32).T
    emb = _run(idx, *tabs)
    cont = x[:, _N_CAT:_N_CAT + _N_CONT]
    return emb, cont
